# baseline (speedup 1.0000x reference)
import jax
import jax.numpy as jnp
from jax.experimental import pallas as pl
from jax.experimental.pallas import tpu as pltpu

_NUM_PARTICLES = 500000
_D = 64
_N = _NUM_PARTICLES * _D
_NSPLIT = 8
_PER = _N // _NSPLIT


def _dma_body(src_hbm, out_hbm, *sems):
    cps = [
        pltpu.make_async_copy(
            src_hbm.at[pl.ds(i * _PER, _PER)],
            out_hbm.at[pl.ds(i * _PER, _PER)],
            sems[i],
        )
        for i in range(_NSPLIT)
    ]
    for cp in cps:
        cp.start()
    for cp in cps:
        cp.wait()


@jax.jit
def kernel(sample, particles):
    del sample
    flat = particles.reshape(_N)
    out = pl.pallas_call(
        _dma_body,
        out_shape=jax.ShapeDtypeStruct((_N,), jnp.float32),
        in_specs=[pl.BlockSpec(memory_space=pltpu.HBM)],
        out_specs=pl.BlockSpec(memory_space=pltpu.HBM),
        scratch_shapes=[pltpu.SemaphoreType.DMA] * _NSPLIT,
    )(flat)
    return out.reshape(_NUM_PARTICLES, _D)


# speedup vs baseline: 6.4236x; 6.4236x over previous
import jax
import jax.numpy as jnp
from jax.experimental import pallas as pl
from jax.experimental.pallas import tpu as pltpu

_NUM_PARTICLES = 500000
_D = 64
_ROWS = _NUM_PARTICLES * _D // 128
_BLOCK = 10000
_GRID = _ROWS // _BLOCK


def _copy_block(src_ref, out_ref):
    out_ref[...] = src_ref[...]


@jax.jit
def kernel(sample, particles):
    del sample
    flat = particles.reshape(_ROWS, 128)
    out = pl.pallas_call(
        _copy_block,
        out_shape=jax.ShapeDtypeStruct((_ROWS, 128), jnp.float32),
        grid=(_GRID,),
        in_specs=[pl.BlockSpec((_BLOCK, 128), lambda i: (i, 0))],
        out_specs=pl.BlockSpec((_BLOCK, 128), lambda i: (i, 0)),
    )(flat)
    return out.reshape(_NUM_PARTICLES, _D)


# speedup vs baseline: 8.9040x; 1.3861x over previous
import jax
import jax.numpy as jnp
from jax.experimental import pallas as pl
from jax.experimental.pallas import tpu as pltpu

_NUM_PARTICLES = 500000
_D = 64
_BLOCK = 20000
_GRID = _NUM_PARTICLES // _BLOCK


def _copy_block(src_ref, out_ref):
    out_ref[...] = src_ref[...]


@jax.jit
def kernel(sample, particles):
    del sample
    return pl.pallas_call(
        _copy_block,
        out_shape=jax.ShapeDtypeStruct((_NUM_PARTICLES, _D), jnp.float32),
        grid=(_GRID,),
        in_specs=[pl.BlockSpec((_BLOCK, _D), lambda i: (i, 0))],
        out_specs=pl.BlockSpec((_BLOCK, _D), lambda i: (i, 0)),
    )(particles)


# speedup vs baseline: 8.9083x; 1.0005x over previous
import jax
import jax.numpy as jnp
from jax.experimental import pallas as pl
from jax.experimental.pallas import tpu as pltpu

_NUM_PARTICLES = 500000
_D = 64
_CHUNK = 10000
_NCH = _NUM_PARTICLES // _CHUNK
_K = 8


def _ring_body(src_hbm, out_hbm, *rest):
    bufs = rest[:_K]
    isems = rest[_K:2 * _K]
    osems = rest[2 * _K:3 * _K]

    def in_cp(i):
        b = i % _K
        return pltpu.make_async_copy(
            src_hbm.at[pl.ds(i * _CHUNK, _CHUNK), :], bufs[b], isems[b])

    def out_cp(i):
        b = i % _K
        return pltpu.make_async_copy(
            bufs[b], out_hbm.at[pl.ds(i * _CHUNK, _CHUNK), :], osems[b])

    for i in range(_NCH):
        if i >= _K:
            out_cp(i - _K).wait()
        in_cp(i).start()
        j = i - (_K - 1)
        if j >= 0:
            in_cp(j).wait()
            out_cp(j).start()
    for j in range(max(0, _NCH - (_K - 1)), _NCH):
        in_cp(j).wait()
        out_cp(j).start()
    for j in range(max(0, _NCH - _K), _NCH):
        out_cp(j).wait()


@jax.jit
def kernel(sample, particles):
    del sample
    return pl.pallas_call(
        _ring_body,
        out_shape=jax.ShapeDtypeStruct((_NUM_PARTICLES, _D), jnp.float32),
        in_specs=[pl.BlockSpec(memory_space=pltpu.HBM)],
        out_specs=pl.BlockSpec(memory_space=pltpu.HBM),
        scratch_shapes=(
            [pltpu.VMEM((_CHUNK, _D), jnp.float32)] * _K
            + [pltpu.SemaphoreType.DMA] * (2 * _K)
        ),
    )(particles)
